# Initial kernel scaffold; baseline (speedup 1.0000x reference)
import functools

import numpy as np
import jax
import jax.numpy as jnp
from jax.experimental import pallas as pl
from jax.experimental.pallas import tpu as pltpu

_K = 9
_NA = 5
_NH = 19
_NW = 19
_NPIX = _NH * _NW
_NB = 32
_NLBL = 2 * _K + 3
_NGT = 50
_CONF0 = float(np.exp(2.0) - 1.0 + 1e-5)
_ANCHORS = [1.482, 2.2412, 2.0501, 3.1265, 2.3946, 4.6891, 3.1018, 3.0157,
            4.5509, 5.9446]
_OBJ_SCALE_SQRT = float(np.sqrt(5.0))
_XCH = [0] + [k + 2 for k in range(1, _K)]
_YCH = [1] + [k + 3 for k in range(1, _K)]


def _sig(x):
    return 1.0 / (1.0 + jnp.exp(-x))


def _loss_body(cur_ref, prev_ref, tgt_ref, out_ref):
    b = pl.program_id(0)
    cur = cur_ref[0]
    prev = prev_ref[0]
    tgt = tgt_ref[0]

    f32 = jnp.float32
    i32 = jnp.int32

    pixi = jax.lax.broadcasted_iota(i32, (1, _NPIX), 1)
    gxpix = (pixi % _NW).astype(f32)
    gypix = (pixi // _NW).astype(f32)

    gx = [tgt[:, 1 + 2 * k:2 + 2 * k] for k in range(_K)]
    gy = [tgt[:, 2 + 2 * k:3 + 2 * k] for k in range(_K)]

    ti = jax.lax.broadcasted_iota(i32, (_NGT, _NGT), 0)
    si = jax.lax.broadcasted_iota(i32, (_NGT, _NGT), 1)
    tril = (si <= ti).astype(f32)
    ind0 = (tgt[:, 1:2] == 0.0).astype(f32)
    zcnt = jax.lax.dot_general(tril, ind0, (((1,), (0,)), ((), ())),
                               preferred_element_type=f32)
    validf = (zcnt == 0.0).astype(f32)

    gi0 = (gx[0] * float(_NW)).astype(i32)
    gj0 = (gy[0] * float(_NH)).astype(i32)
    gi0f = gi0.astype(f32)
    gj0f = gj0.astype(f32)
    q = gj0 * _NW + gi0
    qpos = jax.lax.broadcasted_iota(i32, (_NGT, _NPIX), 1)
    ohq = (qpos == q).astype(f32)

    gw = tgt[:, _NLBL - 2:_NLBL - 1] * float(_NW)
    gh = tgt[:, _NLBL - 1:_NLBL] * float(_NH)
    ious = []
    for n in range(_NA):
        aw = _ANCHORS[2 * n]
        ah = _ANCHORS[2 * n + 1]
        mx = jnp.minimum(-aw / 2.0, -gw / 2.0)
        Mx = jnp.maximum(aw / 2.0, gw / 2.0)
        my = jnp.minimum(-ah / 2.0, -gh / 2.0)
        My = jnp.maximum(ah / 2.0, gh / 2.0)
        cw = aw + gw - (Mx - mx)
        chh = ah + gh - (My - my)
        carea = cw * chh
        uarea = aw * ah + gw * gh - carea
        ious.append(jnp.where((cw <= 0.0) | (chh <= 0.0), 0.0, carea / uarea))
    iouc = jnp.concatenate(ious, axis=1)
    best = jnp.max(iouc, axis=1, keepdims=True)
    nio = jax.lax.broadcasted_iota(i32, (_NGT, _NA), 1)
    first = jnp.min(jnp.where(iouc == best, nio, _NA + 1), axis=1,
                    keepdims=True)
    bn = jnp.where(best > 0.0, first, _NA - 1)

    acc_scale = 1.0 / (9.0 * _CONF0)
    thresh_raw = 0.6
    noobj_vec = jnp.zeros((1, _NPIX), f32)
    G = jnp.zeros((_NGT, 14), f32)
    for a in range(_NA):
        acc = jnp.zeros((_NGT, _NPIX), f32)
        for k in range(_K):
            vx = cur[a * 32 + _XCH[k]:a * 32 + _XCH[k] + 1, :]
            vy = cur[a * 32 + _YCH[k]:a * 32 + _YCH[k] + 1, :]
            if k == 0:
                vx = _sig(vx)
                vy = _sig(vy)
            px = (vx + gxpix) / float(_NW)
            py = (vy + gypix) / float(_NH)
            dx = (gx[k] - px) * 640.0
            dy = (gy[k] - py) * 480.0
            dist = jnp.sqrt(dx * dx + dy * dy)
            acc = acc + jnp.maximum(jnp.exp(2.0 - dist * 0.025) - 1.0, 0.0)
        conf_a = acc * acc_scale
        mc = jnp.max(jnp.where(validf > 0.0, conf_a, 0.0), axis=0,
                     keepdims=True)
        confp_a = _sig(cur[a * 32 + 2 * _K:a * 32 + 2 * _K + 1, :])
        noobj_a = (mc <= thresh_raw).astype(f32)
        noobj_vec = noobj_vec + noobj_a * confp_a * confp_a

        feat = jnp.concatenate(
            [cur[a * 32:a * 32 + 12, :],
             cur[a * 32 + 2 * _K:a * 32 + 2 * _K + 1, :],
             mc], axis=0)
        Ga = jax.lax.dot_general(ohq, feat, (((1,), (1,)), ((), ())),
                                 preferred_element_type=f32)
        G = G + jnp.where(bn == a, 1.0, 0.0) * Ga

    noobj_sum = jnp.sum(noobj_vec)

    P = jax.lax.dot_general(ohq, prev[4 * 32:4 * 32 + 12, :],
                            (((1,), (1,)), ((), ())),
                            preferred_element_type=f32)
    tacc = jnp.zeros((_NGT, 1), f32)
    for k in range(_K):
        vx = P[:, _XCH[k]:_XCH[k] + 1]
        vy = P[:, _YCH[k]:_YCH[k] + 1]
        if k == 0:
            vx = _sig(vx)
            vy = _sig(vy)
        pbx = (vx + gi0f) / float(_NW)
        pby = (vy + gj0f) / float(_NH)
        dxk = (gx[k] - pbx) * 640.0
        dyk = (gy[k] - pby) * 480.0
        dk = jnp.sqrt(dxk * dxk + dyk * dyk)
        tacc = tacc + jnp.maximum(jnp.exp(2.0 - dk * 0.025) - 1.0, 0.0)
    tconf = tacc * (1.0 / (9.0 * _CONF0))

    cellf = (bn * _NPIX + q).astype(f32)
    eyef = (ti == si).astype(f32)
    cell_row = jax.lax.dot_general(cellf, eyef, (((0,), (0,)), ((), ())),
                                   preferred_element_type=f32)
    later_same = ((cellf == cell_row) & (si > ti)).astype(f32)
    kcnt = jax.lax.dot_general(later_same, validf, (((1,), (0,)), ((), ())),
                               preferred_element_type=f32)
    winf = validf * (kcnt == 0.0).astype(f32)

    confp_c = _sig(G[:, 12:13])
    noobj_c = (G[:, 13:14] <= thresh_raw).astype(f32)
    s5 = _OBJ_SCALE_SQRT
    dconf = confp_c * s5 - tconf * s5
    adj = 0.5 * (dconf * dconf - noobj_c * confp_c * confp_c)
    for k in range(_K):
        xc = G[:, _XCH[k]:_XCH[k] + 1]
        yc = G[:, _YCH[k]:_YCH[k] + 1]
        if k == 0:
            xc = _sig(xc)
            yc = _sig(yc)
        tx = gx[k] * float(_NW) - gi0f
        ty = gy[k] * float(_NH) - gj0f
        adj = adj + 0.5 * ((xc - tx) * (xc - tx) + (yc - ty) * (yc - ty))

    loss_b = 0.5 * noobj_sum + jnp.sum(winf * adj)

    @pl.when(b == 0)
    def _():
        out_ref[0, 0] = 0.0

    out_ref[0, 0] = out_ref[0, 0] + loss_b


@functools.partial(jax.jit, static_argnames=("interpret",))
def _region_loss(output, target, interpret=False):
    out_r = output.astype(jnp.float32).reshape(_NB, 160, _NPIX)
    tgt_r = target.astype(jnp.float32).reshape(_NB, _NGT, _NLBL)
    res = pl.pallas_call(
        _loss_body,
        grid=(_NB,),
        in_specs=[
            pl.BlockSpec((1, 160, _NPIX), lambda b: (b, 0, 0)),
            pl.BlockSpec((1, 160, _NPIX), lambda b: ((b + _NB - 1) % _NB, 0, 0)),
            pl.BlockSpec((1, _NGT, _NLBL), lambda b: (b, 0, 0)),
        ],
        out_specs=pl.BlockSpec((1, 1), lambda b: (0, 0),
                               memory_space=pltpu.SMEM),
        out_shape=jax.ShapeDtypeStruct((1, 1), jnp.float32),
        interpret=interpret,
    )(out_r, out_r, tgt_r)
    return res[0, 0]


def kernel(output, target, epoch):
    return _region_loss(output, target)


# speedup vs baseline: 29.7946x; 29.7946x over previous
import functools

import numpy as np
import jax
import jax.numpy as jnp
from jax.experimental import pallas as pl
from jax.experimental.pallas import tpu as pltpu

_K = 9
_NA = 5
_NH = 19
_NW = 19
_NPIX = _NH * _NW
_NB = 32
_NLBL = 2 * _K + 3
_NGT = 50
_CONF0 = float(np.exp(2.0) - 1.0 + 1e-5)
_ANCHORS = [1.482, 2.2412, 2.0501, 3.1265, 2.3946, 4.6891, 3.1018, 3.0157,
            4.5509, 5.9446]
_OBJ_SCALE_SQRT = float(np.sqrt(5.0))
_XCH = [0] + [k + 2 for k in range(1, _K)]
_YCH = [1] + [k + 3 for k in range(1, _K)]


def _sig(x):
    return 1.0 / (1.0 + jnp.exp(-x))


def _loss_body(cur_ref, prev_ref, tgt_ref, out_ref):
    b = pl.program_id(0)
    cur = cur_ref[0]
    prev = prev_ref[0]
    tgt = tgt_ref[0]

    f32 = jnp.float32
    i32 = jnp.int32

    pixi = jax.lax.broadcasted_iota(i32, (1, _NPIX), 1)
    gxpix = (pixi % _NW).astype(f32)
    gypix = (pixi // _NW).astype(f32)

    gx = [tgt[:, 1 + 2 * k:2 + 2 * k] for k in range(_K)]
    gy = [tgt[:, 2 + 2 * k:3 + 2 * k] for k in range(_K)]

    ti = jax.lax.broadcasted_iota(i32, (_NGT, _NGT), 0)
    si = jax.lax.broadcasted_iota(i32, (_NGT, _NGT), 1)
    tril = (si <= ti).astype(f32)
    ind0 = (tgt[:, 1:2] == 0.0).astype(f32)
    zcnt = jax.lax.dot_general(tril, ind0, (((1,), (0,)), ((), ())),
                               preferred_element_type=f32,
                               precision=jax.lax.Precision.HIGHEST)
    validf = (zcnt == 0.0).astype(f32)

    gi0 = (gx[0] * float(_NW)).astype(i32)
    gj0 = (gy[0] * float(_NH)).astype(i32)
    gi0f = gi0.astype(f32)
    gj0f = gj0.astype(f32)
    q = gj0 * _NW + gi0
    qpos = jax.lax.broadcasted_iota(i32, (_NGT, _NPIX), 1)
    ohq = (qpos == q).astype(f32)

    gw = tgt[:, _NLBL - 2:_NLBL - 1] * float(_NW)
    gh = tgt[:, _NLBL - 1:_NLBL] * float(_NH)
    ious = []
    for n in range(_NA):
        aw = _ANCHORS[2 * n]
        ah = _ANCHORS[2 * n + 1]
        mx = jnp.minimum(-aw / 2.0, -gw / 2.0)
        Mx = jnp.maximum(aw / 2.0, gw / 2.0)
        my = jnp.minimum(-ah / 2.0, -gh / 2.0)
        My = jnp.maximum(ah / 2.0, gh / 2.0)
        cw = aw + gw - (Mx - mx)
        chh = ah + gh - (My - my)
        carea = cw * chh
        uarea = aw * ah + gw * gh - carea
        ious.append(jnp.where((cw <= 0.0) | (chh <= 0.0), 0.0, carea / uarea))
    iouc = jnp.concatenate(ious, axis=1)
    best = jnp.max(iouc, axis=1, keepdims=True)
    nio = jax.lax.broadcasted_iota(i32, (_NGT, _NA), 1)
    first = jnp.min(jnp.where(iouc == best, nio, _NA + 1), axis=1,
                    keepdims=True)
    bn = jnp.where(best > 0.0, first, _NA - 1)

    acc_scale = 1.0 / (9.0 * _CONF0)
    thresh_raw = 0.6
    noobj_vec = jnp.zeros((1, _NPIX), f32)
    G = jnp.zeros((_NGT, 14), f32)
    for a in range(_NA):
        acc = jnp.zeros((_NGT, _NPIX), f32)
        for k in range(_K):
            vx = cur[a * 32 + _XCH[k]:a * 32 + _XCH[k] + 1, :]
            vy = cur[a * 32 + _YCH[k]:a * 32 + _YCH[k] + 1, :]
            if k == 0:
                vx = _sig(vx)
                vy = _sig(vy)
            px = (vx + gxpix) / float(_NW)
            py = (vy + gypix) / float(_NH)
            dx = (gx[k] - px) * 640.0
            dy = (gy[k] - py) * 480.0
            dist = jnp.sqrt(dx * dx + dy * dy)
            acc = acc + jnp.maximum(jnp.exp(2.0 - dist * 0.025) - 1.0, 0.0)
        conf_a = acc * acc_scale
        mc = jnp.max(jnp.where(validf > 0.0, conf_a, 0.0), axis=0,
                     keepdims=True)
        confp_a = _sig(cur[a * 32 + 2 * _K:a * 32 + 2 * _K + 1, :])
        noobj_a = (mc <= thresh_raw).astype(f32)
        noobj_vec = noobj_vec + noobj_a * confp_a * confp_a

        feat = jnp.concatenate(
            [cur[a * 32:a * 32 + 12, :],
             cur[a * 32 + 2 * _K:a * 32 + 2 * _K + 1, :],
             mc], axis=0)
        Ga = jax.lax.dot_general(ohq, feat, (((1,), (1,)), ((), ())),
                                 preferred_element_type=f32,
                               precision=jax.lax.Precision.HIGHEST)
        G = G + jnp.where(bn == a, 1.0, 0.0) * Ga

    noobj_sum = jnp.sum(noobj_vec)

    P = jax.lax.dot_general(ohq, prev[4 * 32:4 * 32 + 12, :],
                            (((1,), (1,)), ((), ())),
                            preferred_element_type=f32,
                               precision=jax.lax.Precision.HIGHEST)
    tacc = jnp.zeros((_NGT, 1), f32)
    for k in range(_K):
        vx = P[:, _XCH[k]:_XCH[k] + 1]
        vy = P[:, _YCH[k]:_YCH[k] + 1]
        if k == 0:
            vx = _sig(vx)
            vy = _sig(vy)
        pbx = (vx + gi0f) / float(_NW)
        pby = (vy + gj0f) / float(_NH)
        dxk = (gx[k] - pbx) * 640.0
        dyk = (gy[k] - pby) * 480.0
        dk = jnp.sqrt(dxk * dxk + dyk * dyk)
        tacc = tacc + jnp.maximum(jnp.exp(2.0 - dk * 0.025) - 1.0, 0.0)
    tconf = tacc * (1.0 / (9.0 * _CONF0))

    cellf = (bn * _NPIX + q).astype(f32)
    eyef = (ti == si).astype(f32)
    cell_row = jax.lax.dot_general(cellf, eyef, (((0,), (0,)), ((), ())),
                                   preferred_element_type=f32,
                               precision=jax.lax.Precision.HIGHEST)
    later_same = ((cellf == cell_row) & (si > ti)).astype(f32)
    kcnt = jax.lax.dot_general(later_same, validf, (((1,), (0,)), ((), ())),
                               preferred_element_type=f32,
                               precision=jax.lax.Precision.HIGHEST)
    winf = validf * (kcnt == 0.0).astype(f32)

    confp_c = _sig(G[:, 12:13])
    noobj_c = (G[:, 13:14] <= thresh_raw).astype(f32)
    s5 = _OBJ_SCALE_SQRT
    dconf = confp_c * s5 - tconf * s5
    adj = 0.5 * (dconf * dconf - noobj_c * confp_c * confp_c)
    for k in range(_K):
        xc = G[:, _XCH[k]:_XCH[k] + 1]
        yc = G[:, _YCH[k]:_YCH[k] + 1]
        if k == 0:
            xc = _sig(xc)
            yc = _sig(yc)
        tx = gx[k] * float(_NW) - gi0f
        ty = gy[k] * float(_NH) - gj0f
        adj = adj + 0.5 * ((xc - tx) * (xc - tx) + (yc - ty) * (yc - ty))

    loss_b = 0.5 * noobj_sum + jnp.sum(winf * adj)

    @pl.when(b == 0)
    def _():
        out_ref[0, 0] = 0.0

    out_ref[0, 0] = out_ref[0, 0] + loss_b


@functools.partial(jax.jit, static_argnames=("interpret",))
def _region_loss(output, target, interpret=False):
    out_r = output.astype(jnp.float32).reshape(_NB, 160, _NPIX)
    tgt_r = target.astype(jnp.float32).reshape(_NB, _NGT, _NLBL)
    res = pl.pallas_call(
        _loss_body,
        grid=(_NB,),
        in_specs=[
            pl.BlockSpec((1, 160, _NPIX), lambda b: (b, 0, 0)),
            pl.BlockSpec((1, 160, _NPIX), lambda b: ((b + _NB - 1) % _NB, 0, 0)),
            pl.BlockSpec((1, _NGT, _NLBL), lambda b: (b, 0, 0)),
        ],
        out_specs=pl.BlockSpec((1, 1), lambda b: (0, 0),
                               memory_space=pltpu.SMEM),
        out_shape=jax.ShapeDtypeStruct((1, 1), jnp.float32),
        interpret=interpret,
    )(out_r, out_r, tgt_r)
    return res[0, 0]


def kernel(output, target, epoch):
    return _region_loss(output, target)


# speedup vs baseline: 37.5073x; 1.2589x over previous
import functools

import numpy as np
import jax
import jax.numpy as jnp
from jax.experimental import pallas as pl
from jax.experimental.pallas import tpu as pltpu

_K = 9
_NA = 5
_NH = 19
_NW = 19
_NPIX = _NH * _NW
_NB = 32
_NLBL = 2 * _K + 3
_NGT = 50
_CONF0 = float(np.exp(2.0) - 1.0 + 1e-5)
_ANCHORS = [1.482, 2.2412, 2.0501, 3.1265, 2.3946, 4.6891, 3.1018, 3.0157,
            4.5509, 5.9446]
_OBJ_SCALE_SQRT = float(np.sqrt(5.0))
_XCH = [0] + [k + 2 for k in range(1, _K)]
_YCH = [1] + [k + 3 for k in range(1, _K)]


def _sig(x):
    return 1.0 / (1.0 + jnp.exp(-x))


def _loss_body(cur_ref, prev_ref, tgt_ref, out_ref):
    b = pl.program_id(0)
    cur = cur_ref[0]
    prev = prev_ref[0]
    tgt = tgt_ref[0]

    f32 = jnp.float32
    i32 = jnp.int32

    pixi = jax.lax.broadcasted_iota(i32, (1, _NPIX), 1)
    gxpix = (pixi % _NW).astype(f32)
    gypix = (pixi // _NW).astype(f32)

    gx = [tgt[:, 1 + 2 * k:2 + 2 * k] for k in range(_K)]
    gy = [tgt[:, 2 + 2 * k:3 + 2 * k] for k in range(_K)]

    ti = jax.lax.broadcasted_iota(i32, (_NGT, _NGT), 0)
    si = jax.lax.broadcasted_iota(i32, (_NGT, _NGT), 1)
    tril = (si <= ti).astype(f32)
    ind0 = (tgt[:, 1:2] == 0.0).astype(f32)
    zcnt = jax.lax.dot_general(tril, ind0, (((1,), (0,)), ((), ())),
                               preferred_element_type=f32,
                               precision=jax.lax.Precision.HIGHEST)
    validf = (zcnt == 0.0).astype(f32)

    gi0 = (gx[0] * float(_NW)).astype(i32)
    gj0 = (gy[0] * float(_NH)).astype(i32)
    gi0f = gi0.astype(f32)
    gj0f = gj0.astype(f32)
    q = gj0 * _NW + gi0
    qpos = jax.lax.broadcasted_iota(i32, (_NGT, _NPIX), 1)
    ohq = (qpos == q).astype(f32)

    gw = tgt[:, _NLBL - 2:_NLBL - 1] * float(_NW)
    gh = tgt[:, _NLBL - 1:_NLBL] * float(_NH)
    ious = []
    for n in range(_NA):
        aw = _ANCHORS[2 * n]
        ah = _ANCHORS[2 * n + 1]
        mx = jnp.minimum(-aw / 2.0, -gw / 2.0)
        Mx = jnp.maximum(aw / 2.0, gw / 2.0)
        my = jnp.minimum(-ah / 2.0, -gh / 2.0)
        My = jnp.maximum(ah / 2.0, gh / 2.0)
        cw = aw + gw - (Mx - mx)
        chh = ah + gh - (My - my)
        carea = cw * chh
        uarea = aw * ah + gw * gh - carea
        ious.append(jnp.where((cw <= 0.0) | (chh <= 0.0), 0.0, carea / uarea))
    iouc = jnp.concatenate(ious, axis=1)
    best = jnp.max(iouc, axis=1, keepdims=True)
    nio = jax.lax.broadcasted_iota(i32, (_NGT, _NA), 1)
    first = jnp.min(jnp.where(iouc == best, nio, _NA + 1), axis=1,
                    keepdims=True)
    bn = jnp.where(best > 0.0, first, _NA - 1)

    c2 = 0.025 * float(np.log2(np.e))
    c1 = 2.0 * float(np.log2(np.e))
    thr_raw = 0.6 * 9.0 * _CONF0
    gxs = [g * (640.0 * c2) for g in gx]
    gys = [g * (480.0 * c2) for g in gy]
    noobj_vec = jnp.zeros((1, _NPIX), f32)
    G = jnp.zeros((_NGT, 14), f32)
    for a in range(_NA):
        acc = jnp.zeros((_NGT, _NPIX), f32)
        for k in range(_K):
            vx = cur[a * 32 + _XCH[k]:a * 32 + _XCH[k] + 1, :]
            vy = cur[a * 32 + _YCH[k]:a * 32 + _YCH[k] + 1, :]
            if k == 0:
                vx = _sig(vx)
                vy = _sig(vy)
            hx = (vx + gxpix) * (640.0 * c2 / float(_NW))
            hy = (vy + gypix) * (480.0 * c2 / float(_NH))
            dx = gxs[k] - hx
            dy = gys[k] - hy
            d2 = jnp.maximum(dx * dx + dy * dy, 1e-24)
            arg = c1 - d2 * jax.lax.rsqrt(d2)
            acc = acc + jnp.maximum(jnp.exp2(arg) - 1.0, 0.0)
        mc = jnp.max(acc * validf, axis=0, keepdims=True)
        confp_a = _sig(cur[a * 32 + 2 * _K:a * 32 + 2 * _K + 1, :])
        noobj_a = (mc <= thr_raw).astype(f32)
        noobj_vec = noobj_vec + noobj_a * confp_a * confp_a

        feat = jnp.concatenate(
            [cur[a * 32:a * 32 + 12, :],
             cur[a * 32 + 2 * _K:a * 32 + 2 * _K + 1, :],
             mc], axis=0)
        Ga = jax.lax.dot_general(ohq, feat, (((1,), (1,)), ((), ())),
                                 preferred_element_type=f32,
                               precision=jax.lax.Precision.HIGHEST)
        G = G + jnp.where(bn == a, 1.0, 0.0) * Ga

    noobj_sum = jnp.sum(noobj_vec)

    P = jax.lax.dot_general(ohq, prev[4 * 32:4 * 32 + 12, :],
                            (((1,), (1,)), ((), ())),
                            preferred_element_type=f32,
                               precision=jax.lax.Precision.HIGHEST)
    tacc = jnp.zeros((_NGT, 1), f32)
    for k in range(_K):
        vx = P[:, _XCH[k]:_XCH[k] + 1]
        vy = P[:, _YCH[k]:_YCH[k] + 1]
        if k == 0:
            vx = _sig(vx)
            vy = _sig(vy)
        pbx = (vx + gi0f) / float(_NW)
        pby = (vy + gj0f) / float(_NH)
        dxk = (gx[k] - pbx) * 640.0
        dyk = (gy[k] - pby) * 480.0
        dk = jnp.sqrt(dxk * dxk + dyk * dyk)
        tacc = tacc + jnp.maximum(jnp.exp(2.0 - dk * 0.025) - 1.0, 0.0)
    tconf = tacc * (1.0 / (9.0 * _CONF0))

    cellf = (bn * _NPIX + q).astype(f32)
    eyef = (ti == si).astype(f32)
    cell_row = jax.lax.dot_general(cellf, eyef, (((0,), (0,)), ((), ())),
                                   preferred_element_type=f32,
                               precision=jax.lax.Precision.HIGHEST)
    later_same = ((cellf == cell_row) & (si > ti)).astype(f32)
    kcnt = jax.lax.dot_general(later_same, validf, (((1,), (0,)), ((), ())),
                               preferred_element_type=f32,
                               precision=jax.lax.Precision.HIGHEST)
    winf = validf * (kcnt == 0.0).astype(f32)

    confp_c = _sig(G[:, 12:13])
    noobj_c = (G[:, 13:14] <= thr_raw).astype(f32)
    s5 = _OBJ_SCALE_SQRT
    dconf = confp_c * s5 - tconf * s5
    adj = 0.5 * (dconf * dconf - noobj_c * confp_c * confp_c)
    for k in range(_K):
        xc = G[:, _XCH[k]:_XCH[k] + 1]
        yc = G[:, _YCH[k]:_YCH[k] + 1]
        if k == 0:
            xc = _sig(xc)
            yc = _sig(yc)
        tx = gx[k] * float(_NW) - gi0f
        ty = gy[k] * float(_NH) - gj0f
        adj = adj + 0.5 * ((xc - tx) * (xc - tx) + (yc - ty) * (yc - ty))

    loss_b = 0.5 * noobj_sum + jnp.sum(winf * adj)

    @pl.when(b == 0)
    def _():
        out_ref[0, 0] = 0.0

    out_ref[0, 0] = out_ref[0, 0] + loss_b


@functools.partial(jax.jit, static_argnames=("interpret",))
def _region_loss(output, target, interpret=False):
    out_r = output.astype(jnp.float32).reshape(_NB, 160, _NPIX)
    tgt_r = target.astype(jnp.float32).reshape(_NB, _NGT, _NLBL)
    res = pl.pallas_call(
        _loss_body,
        grid=(_NB,),
        in_specs=[
            pl.BlockSpec((1, 160, _NPIX), lambda b: (b, 0, 0)),
            pl.BlockSpec((1, 160, _NPIX), lambda b: ((b + _NB - 1) % _NB, 0, 0)),
            pl.BlockSpec((1, _NGT, _NLBL), lambda b: (b, 0, 0)),
        ],
        out_specs=pl.BlockSpec((1, 1), lambda b: (0, 0),
                               memory_space=pltpu.SMEM),
        out_shape=jax.ShapeDtypeStruct((1, 1), jnp.float32),
        interpret=interpret,
    )(out_r, out_r, tgt_r)
    return res[0, 0]


def kernel(output, target, epoch):
    return _region_loss(output, target)
